# baseline (device time: 200818 ns/iter reference)
import jax
import jax.numpy as jnp
from jax import lax
from jax.experimental import pallas as pl
from jax.experimental.pallas import tpu as pltpu

Y = 4
V_PER = 16384
K = 16


def kernel(ids, E):
    t = ids.shape[0]
    v_per, d = E.shape
    rows = t // K
    d2 = d // 2

    my_y = lax.axis_index("y")
    local = (ids - my_y * v_per).astype(jnp.int32)
    valid = (local >= 0) & (local < v_per)
    mask = valid.astype(jnp.float32)[:, None]
    counts = valid.reshape(K, rows).sum(axis=1).astype(jnp.int32)
    tok = jnp.nonzero(valid, size=t, fill_value=0)[0].astype(jnp.int32)
    rowc = local[tok]

    def body(tok_ref, rowc_ref, counts_ref, mask_ref, e_ref, out_ref,
             part_ref, lo_ref, hi_ref, sbuf_ref,
             gather_sems, lo_sems, hi_sems, fin_sems, xfin_sems, s_sems):
        my_x = lax.axis_index("x")
        yy = lax.axis_index("y")
        my_z = lax.axis_index("z")
        ox = 1 - my_x
        mycol = pl.ds(my_x * d2, d2)
        ocol = pl.ds(ox * d2, d2)

        total = counts_ref[0]
        for c in range(1, K):
            total = total + counts_ref[c]

        def issue(j, _):
            tk = tok_ref[j]
            pltpu.make_async_copy(
                e_ref.at[rowc_ref[j], mycol], part_ref.at[tk],
                gather_sems.at[tk // rows]
            ).start()
            return 0

        lax.fori_loop(0, total, issue, 0)

        barrier_sem = pltpu.get_barrier_semaphore()
        lo_nbr = jnp.maximum(yy - 1, 0)
        hi_nbr = jnp.minimum(yy + 1, Y - 1)

        pl.semaphore_signal(
            barrier_sem, inc=1, device_id=(ox, yy, my_z),
            device_id_type=pl.DeviceIdType.MESH,
        )

        @pl.when(yy > 0)
        def _():
            pl.semaphore_signal(
                barrier_sem, inc=1, device_id=(my_x, lo_nbr, my_z),
                device_id_type=pl.DeviceIdType.MESH,
            )

        @pl.when(yy < Y - 1)
        def _():
            pl.semaphore_signal(
                barrier_sem, inc=1, device_id=(my_x, hi_nbr, my_z),
                device_id_type=pl.DeviceIdType.MESH,
            )

        n_nbrs = 1 + (yy > 0).astype(jnp.int32) + (yy < Y - 1).astype(jnp.int32)
        pl.semaphore_wait(barrier_sem, n_nbrs)

        def rc(c):
            return pl.ds(c * rows, rows)

        def ensure(c):
            def w(j, _):
                pltpu.make_async_copy(
                    e_ref.at[0, mycol], part_ref.at[0], gather_sems.at[c]
                ).wait()
                return 0

            lax.fori_loop(0, counts_ref[c], w, 0)
            part_ref[rc(c)] = jnp.where(
                mask_ref[rc(c)] != 0.0, part_ref[rc(c)], 0.0
            )

        def desc(src, dst, ssem, rsem, dev):
            return pltpu.make_async_remote_copy(
                src_ref=src, dst_ref=dst, send_sem=ssem, recv_sem=rsem,
                device_id=dev, device_id_type=pl.DeviceIdType.MESH,
            )

        def xdesc_send(c):
            return desc(out_ref.at[rc(c), mycol], out_ref.at[rc(c), mycol],
                        s_sems.at[2, c], xfin_sems.at[c], (ox, yy, my_z))

        def edge(inner_y, inbuf, insems):
            def _():
                started = []
                for c in range(K):
                    ensure(c)
                    s = desc(part_ref.at[rc(c)], inbuf.at[c],
                             s_sems.at[0, c], insems.at[c],
                             (my_x, inner_y, my_z))
                    s.start()
                    started.append(s)
                for c in range(K):
                    desc(part_ref.at[rc(c)], out_ref.at[rc(c), mycol],
                         s_sems.at[1, c], fin_sems.at[c],
                         (my_x, inner_y, my_z)).wait_recv()
                    xs = xdesc_send(c)
                    xs.start()
                    started.append(xs)
                for c in range(K):
                    desc(out_ref.at[rc(c), ocol], out_ref.at[rc(c), ocol],
                         s_sems.at[1, c], xfin_sems.at[c],
                         (ox, yy, my_z)).wait_recv()
                for s in started:
                    s.wait_send()
            return _

        def middle(outer_y, other_y, mybuf, mysems, otherbuf, othersems):
            def _():
                started = []
                for c in range(K):
                    ensure(c)
                    desc(part_ref.at[rc(c)], mybuf.at[c],
                         s_sems.at[0, c], mysems.at[c],
                         (my_x, outer_y, my_z)).wait_recv()
                    sbuf_ref[c] = part_ref[rc(c)] + mybuf[c]
                    s = desc(sbuf_ref.at[c], mybuf.at[c],
                             s_sems.at[0, c], mysems.at[c],
                             (my_x, other_y, my_z))
                    s.start()
                    started.append(s)
                    desc(part_ref.at[rc(c)], otherbuf.at[c],
                         s_sems.at[1, c], othersems.at[c],
                         (my_x, other_y, my_z)).wait_recv()
                    out_ref[rc(c), mycol] = sbuf_ref[c] + otherbuf[c]
                    f = desc(out_ref.at[rc(c), mycol], out_ref.at[rc(c), mycol],
                             s_sems.at[1, c], fin_sems.at[c],
                             (my_x, outer_y, my_z))
                    f.start()
                    started.append(f)
                    xs = xdesc_send(c)
                    xs.start()
                    started.append(xs)
                for c in range(K):
                    desc(out_ref.at[rc(c), ocol], out_ref.at[rc(c), ocol],
                         s_sems.at[0, c], xfin_sems.at[c],
                         (ox, yy, my_z)).wait_recv()
                for s in started:
                    s.wait_send()
            return _

        pl.when(yy == 0)(edge(1, lo_ref, lo_sems))
        pl.when(yy == 3)(edge(2, hi_ref, hi_sems))
        pl.when(yy == 1)(middle(0, 2, lo_ref, lo_sems, hi_ref, hi_sems))
        pl.when(yy == 2)(middle(3, 1, hi_ref, hi_sems, lo_ref, lo_sems))

    return pl.pallas_call(
        body,
        out_shape=jax.ShapeDtypeStruct((t, d), jnp.float32),
        in_specs=[
            pl.BlockSpec(memory_space=pltpu.SMEM),
            pl.BlockSpec(memory_space=pltpu.SMEM),
            pl.BlockSpec(memory_space=pltpu.SMEM),
            pl.BlockSpec(memory_space=pltpu.VMEM),
            pl.BlockSpec(memory_space=pl.ANY),
        ],
        out_specs=pl.BlockSpec(memory_space=pltpu.VMEM),
        scratch_shapes=[
            pltpu.VMEM((t, d2), jnp.float32),
            pltpu.VMEM((K, rows, d2), jnp.float32),
            pltpu.VMEM((K, rows, d2), jnp.float32),
            pltpu.VMEM((K, rows, d2), jnp.float32),
            pltpu.SemaphoreType.DMA((K,)),
            pltpu.SemaphoreType.DMA((K,)),
            pltpu.SemaphoreType.DMA((K,)),
            pltpu.SemaphoreType.DMA((K,)),
            pltpu.SemaphoreType.DMA((K,)),
            pltpu.SemaphoreType.DMA((3, K)),
        ],
        compiler_params=pltpu.CompilerParams(collective_id=0),
    )(tok, rowc, counts, mask, E)


# device time: 121814 ns/iter; 1.6486x vs baseline; 1.6486x over previous
import jax
import jax.numpy as jnp
from jax import lax
from jax.experimental import pallas as pl
from jax.experimental.pallas import tpu as pltpu

Y = 4
V_PER = 16384
K = 16


def kernel(ids, E):
    t = ids.shape[0]
    v_per, d = E.shape
    rows = t // K
    d2 = d // 2

    my_y = lax.axis_index("y")
    local = (ids - my_y * v_per).astype(jnp.int32)
    valid = (local >= 0) & (local < v_per)
    mask = valid.astype(jnp.float32)[:, None]
    counts = valid.reshape(K, rows).sum(axis=1).astype(jnp.int32)

    def body(lraw_ref, counts_ref, mask_ref, e_ref, out_ref,
             part_ref, lo_ref, hi_ref, sbuf_ref, tok_ref, rowc_ref,
             gather_sems, lo_sems, hi_sems, fin_sems, xfin_sems, s_sems):
        my_x = lax.axis_index("x")
        yy = lax.axis_index("y")
        my_z = lax.axis_index("z")
        ox = 1 - my_x
        mycol = pl.ds(my_x * d2, d2)
        ocol = pl.ds(ox * d2, d2)

        def compact(i, cnt):
            r = lraw_ref[i]
            tok_ref[cnt] = i
            rowc_ref[cnt] = r
            owned = (r >= 0) & (r < v_per)
            return cnt + owned.astype(jnp.int32)

        total = lax.fori_loop(0, t, compact, jnp.int32(0), unroll=8)

        def issue(j, _):
            tk = tok_ref[j]
            pltpu.make_async_copy(
                e_ref.at[rowc_ref[j], mycol], part_ref.at[tk],
                gather_sems.at[tk // rows]
            ).start()
            return 0

        lax.fori_loop(0, total, issue, 0)

        barrier_sem = pltpu.get_barrier_semaphore()
        lo_nbr = jnp.maximum(yy - 1, 0)
        hi_nbr = jnp.minimum(yy + 1, Y - 1)

        pl.semaphore_signal(
            barrier_sem, inc=1, device_id=(ox, yy, my_z),
            device_id_type=pl.DeviceIdType.MESH,
        )

        @pl.when(yy > 0)
        def _():
            pl.semaphore_signal(
                barrier_sem, inc=1, device_id=(my_x, lo_nbr, my_z),
                device_id_type=pl.DeviceIdType.MESH,
            )

        @pl.when(yy < Y - 1)
        def _():
            pl.semaphore_signal(
                barrier_sem, inc=1, device_id=(my_x, hi_nbr, my_z),
                device_id_type=pl.DeviceIdType.MESH,
            )

        n_nbrs = 1 + (yy > 0).astype(jnp.int32) + (yy < Y - 1).astype(jnp.int32)
        pl.semaphore_wait(barrier_sem, n_nbrs)

        def rc(c):
            return pl.ds(c * rows, rows)

        def ensure(c):
            def w(j, _):
                pltpu.make_async_copy(
                    e_ref.at[0, mycol], part_ref.at[0], gather_sems.at[c]
                ).wait()
                return 0

            lax.fori_loop(0, counts_ref[c], w, 0)
            part_ref[rc(c)] = jnp.where(
                mask_ref[rc(c)] != 0.0, part_ref[rc(c)], 0.0
            )

        def desc(src, dst, ssem, rsem, dev):
            return pltpu.make_async_remote_copy(
                src_ref=src, dst_ref=dst, send_sem=ssem, recv_sem=rsem,
                device_id=dev, device_id_type=pl.DeviceIdType.MESH,
            )

        def xdesc_send(c):
            return desc(out_ref.at[rc(c), mycol], out_ref.at[rc(c), mycol],
                        s_sems.at[2, c], xfin_sems.at[c], (ox, yy, my_z))

        def edge(inner_y, inbuf, insems):
            def _():
                started = []
                for c in range(K):
                    ensure(c)
                    s = desc(part_ref.at[rc(c)], inbuf.at[c],
                             s_sems.at[0, c], insems.at[c],
                             (my_x, inner_y, my_z))
                    s.start()
                    started.append(s)
                for c in range(K):
                    desc(part_ref.at[rc(c)], out_ref.at[rc(c), mycol],
                         s_sems.at[1, c], fin_sems.at[c],
                         (my_x, inner_y, my_z)).wait_recv()
                    xs = xdesc_send(c)
                    xs.start()
                    started.append(xs)
                for c in range(K):
                    desc(out_ref.at[rc(c), ocol], out_ref.at[rc(c), ocol],
                         s_sems.at[1, c], xfin_sems.at[c],
                         (ox, yy, my_z)).wait_recv()
                for s in started:
                    s.wait_send()
            return _

        def middle(outer_y, other_y, mybuf, mysems, otherbuf, othersems):
            def _():
                started = []
                for c in range(K):
                    ensure(c)
                    desc(part_ref.at[rc(c)], mybuf.at[c],
                         s_sems.at[0, c], mysems.at[c],
                         (my_x, outer_y, my_z)).wait_recv()
                    sbuf_ref[c] = part_ref[rc(c)] + mybuf[c]
                    s = desc(sbuf_ref.at[c], mybuf.at[c],
                             s_sems.at[0, c], mysems.at[c],
                             (my_x, other_y, my_z))
                    s.start()
                    started.append(s)
                    desc(part_ref.at[rc(c)], otherbuf.at[c],
                         s_sems.at[1, c], othersems.at[c],
                         (my_x, other_y, my_z)).wait_recv()
                    out_ref[rc(c), mycol] = sbuf_ref[c] + otherbuf[c]
                    f = desc(out_ref.at[rc(c), mycol], out_ref.at[rc(c), mycol],
                             s_sems.at[1, c], fin_sems.at[c],
                             (my_x, outer_y, my_z))
                    f.start()
                    started.append(f)
                    xs = xdesc_send(c)
                    xs.start()
                    started.append(xs)
                for c in range(K):
                    desc(out_ref.at[rc(c), ocol], out_ref.at[rc(c), ocol],
                         s_sems.at[0, c], xfin_sems.at[c],
                         (ox, yy, my_z)).wait_recv()
                for s in started:
                    s.wait_send()
            return _

        pl.when(yy == 0)(edge(1, lo_ref, lo_sems))
        pl.when(yy == 3)(edge(2, hi_ref, hi_sems))
        pl.when(yy == 1)(middle(0, 2, lo_ref, lo_sems, hi_ref, hi_sems))
        pl.when(yy == 2)(middle(3, 1, hi_ref, hi_sems, lo_ref, lo_sems))

    return pl.pallas_call(
        body,
        out_shape=jax.ShapeDtypeStruct((t, d), jnp.float32),
        in_specs=[
            pl.BlockSpec(memory_space=pltpu.SMEM),
            pl.BlockSpec(memory_space=pltpu.SMEM),
            pl.BlockSpec(memory_space=pltpu.VMEM),
            pl.BlockSpec(memory_space=pl.ANY),
        ],
        out_specs=pl.BlockSpec(memory_space=pltpu.VMEM),
        scratch_shapes=[
            pltpu.VMEM((t, d2), jnp.float32),
            pltpu.VMEM((K, rows, d2), jnp.float32),
            pltpu.VMEM((K, rows, d2), jnp.float32),
            pltpu.VMEM((K, rows, d2), jnp.float32),
            pltpu.SMEM((t,), jnp.int32),
            pltpu.SMEM((t,), jnp.int32),
            pltpu.SemaphoreType.DMA((K,)),
            pltpu.SemaphoreType.DMA((K,)),
            pltpu.SemaphoreType.DMA((K,)),
            pltpu.SemaphoreType.DMA((K,)),
            pltpu.SemaphoreType.DMA((K,)),
            pltpu.SemaphoreType.DMA((3, K)),
        ],
        compiler_params=pltpu.CompilerParams(collective_id=0),
    )(local, counts, mask, E)


# device time: 35615 ns/iter; 5.6386x vs baseline; 3.4203x over previous
import jax
import jax.numpy as jnp
from jax import lax
from jax.experimental import pallas as pl
from jax.experimental.pallas import tpu as pltpu

Y = 4
V_PER = 16384
K = 16


def kernel(ids, E):
    t = ids.shape[0]
    v_per, d = E.shape
    rows = t // K
    d2 = d // 2

    my_y = lax.axis_index("y")
    local = (ids - my_y * v_per).astype(jnp.int32)
    valid = (local >= 0) & (local < v_per)
    mask = valid.astype(jnp.float32)[:, None]
    counts = valid.reshape(K, rows).sum(axis=1).astype(jnp.int32)

    def body(lraw_ref, counts_ref, mask_ref, e_ref, out_ref,
             part_ref, lo_ref, hi_ref, sbuf_ref, tok_ref, rowc_ref,
             gather_sems, lo_sems, hi_sems, fin_sems, xfin_sems, s_sems):
        my_x = lax.axis_index("x")
        yy = lax.axis_index("y")
        my_z = lax.axis_index("z")
        ox = 1 - my_x
        mycol = pl.ds(my_x * d2, d2)
        ocol = pl.ds(ox * d2, d2)

        def compact(i, cnt):
            r = lraw_ref[i]
            tok_ref[cnt] = i
            rowc_ref[cnt] = r
            owned = (r >= 0) & (r < v_per)
            return cnt + owned.astype(jnp.int32)

        total = lax.fori_loop(0, t, compact, jnp.int32(0), unroll=8)

        def issue(j, _):
            tk = tok_ref[j]
            pltpu.make_async_copy(
                e_ref.at[rowc_ref[j], mycol], part_ref.at[tk],
                gather_sems.at[tk // rows]
            ).start()
            return 0

        lax.fori_loop(0, total, issue, 0)

        barrier_sem = pltpu.get_barrier_semaphore()
        lo_nbr = jnp.maximum(yy - 1, 0)
        hi_nbr = jnp.minimum(yy + 1, Y - 1)

        pl.semaphore_signal(
            barrier_sem, inc=1, device_id=(ox, yy, my_z),
            device_id_type=pl.DeviceIdType.MESH,
        )

        @pl.when(yy > 0)
        def _():
            pl.semaphore_signal(
                barrier_sem, inc=1, device_id=(my_x, lo_nbr, my_z),
                device_id_type=pl.DeviceIdType.MESH,
            )

        @pl.when(yy < Y - 1)
        def _():
            pl.semaphore_signal(
                barrier_sem, inc=1, device_id=(my_x, hi_nbr, my_z),
                device_id_type=pl.DeviceIdType.MESH,
            )

        n_nbrs = 1 + (yy > 0).astype(jnp.int32) + (yy < Y - 1).astype(jnp.int32)
        pl.semaphore_wait(barrier_sem, n_nbrs)

        def rc(c):
            return pl.ds(c * rows, rows)

        def ensure(c):
            def w(j, _):
                pltpu.make_async_copy(
                    e_ref.at[0, mycol], part_ref.at[0], gather_sems.at[c]
                ).wait()
                return 0

            lax.fori_loop(0, counts_ref[c], w, 0)
            part_ref[rc(c)] = jnp.where(
                mask_ref[rc(c)] != 0.0, part_ref[rc(c)], 0.0
            )

        def desc(src, dst, ssem, rsem, dev):
            return pltpu.make_async_remote_copy(
                src_ref=src, dst_ref=dst, send_sem=ssem, recv_sem=rsem,
                device_id=dev, device_id_type=pl.DeviceIdType.MESH,
            )

        def xdesc_send(c):
            return desc(out_ref.at[rc(c), mycol], out_ref.at[rc(c), mycol],
                        s_sems.at[2, c], xfin_sems.at[c], (ox, yy, my_z))

        def edge(inner_y, inbuf, insems):
            def _():
                started = []
                for c in range(K):
                    ensure(c)
                    s = desc(part_ref.at[rc(c)], inbuf.at[c],
                             s_sems.at[0, c], insems.at[c],
                             (my_x, inner_y, my_z))
                    s.start()
                    started.append(s)
                for c in range(K):
                    desc(part_ref.at[rc(c)], out_ref.at[rc(c), mycol],
                         s_sems.at[1, c], fin_sems.at[c],
                         (my_x, inner_y, my_z)).wait_recv()
                    xs = xdesc_send(c)
                    xs.start()
                    started.append(xs)
                for c in range(K):
                    desc(out_ref.at[rc(c), ocol], out_ref.at[rc(c), ocol],
                         s_sems.at[1, c], xfin_sems.at[c],
                         (ox, yy, my_z)).wait_recv()
                for s in started:
                    s.wait_send()
            return _

        def middle(outer_y, other_y, mybuf, mysems, otherbuf, othersems):
            def _():
                started = []
                for c in range(K):
                    ensure(c)
                    desc(part_ref.at[rc(c)], mybuf.at[c],
                         s_sems.at[0, c], mysems.at[c],
                         (my_x, outer_y, my_z)).wait_recv()
                    sbuf_ref[c] = part_ref[rc(c)] + mybuf[c]
                    s = desc(sbuf_ref.at[c], mybuf.at[c],
                             s_sems.at[0, c], mysems.at[c],
                             (my_x, other_y, my_z))
                    s.start()
                    started.append(s)
                    desc(part_ref.at[rc(c)], otherbuf.at[c],
                         s_sems.at[1, c], othersems.at[c],
                         (my_x, other_y, my_z)).wait_recv()
                    out_ref[rc(c), mycol] = sbuf_ref[c] + otherbuf[c]
                    f = desc(out_ref.at[rc(c), mycol], out_ref.at[rc(c), mycol],
                             s_sems.at[1, c], fin_sems.at[c],
                             (my_x, outer_y, my_z))
                    f.start()
                    started.append(f)
                    xs = xdesc_send(c)
                    xs.start()
                    started.append(xs)
                for c in range(K):
                    desc(out_ref.at[rc(c), ocol], out_ref.at[rc(c), ocol],
                         s_sems.at[0, c], xfin_sems.at[c],
                         (ox, yy, my_z)).wait_recv()
                for s in started:
                    s.wait_send()
            return _

        import os
        if os.environ.get("PROBE") == "gather":
            for c in range(K):
                ensure(c)
                out_ref[rc(c), mycol] = part_ref[rc(c)]
                out_ref[rc(c), ocol] = part_ref[rc(c)]
            return

        pl.when(yy == 0)(edge(1, lo_ref, lo_sems))
        pl.when(yy == 3)(edge(2, hi_ref, hi_sems))
        pl.when(yy == 1)(middle(0, 2, lo_ref, lo_sems, hi_ref, hi_sems))
        pl.when(yy == 2)(middle(3, 1, hi_ref, hi_sems, lo_ref, lo_sems))

    return pl.pallas_call(
        body,
        out_shape=jax.ShapeDtypeStruct((t, d), jnp.float32),
        in_specs=[
            pl.BlockSpec(memory_space=pltpu.SMEM),
            pl.BlockSpec(memory_space=pltpu.SMEM),
            pl.BlockSpec(memory_space=pltpu.VMEM),
            pl.BlockSpec(memory_space=pl.ANY),
        ],
        out_specs=pl.BlockSpec(memory_space=pltpu.VMEM),
        scratch_shapes=[
            pltpu.VMEM((t, d2), jnp.float32),
            pltpu.VMEM((K, rows, d2), jnp.float32),
            pltpu.VMEM((K, rows, d2), jnp.float32),
            pltpu.VMEM((K, rows, d2), jnp.float32),
            pltpu.SMEM((t,), jnp.int32),
            pltpu.SMEM((t,), jnp.int32),
            pltpu.SemaphoreType.DMA((K,)),
            pltpu.SemaphoreType.DMA((K,)),
            pltpu.SemaphoreType.DMA((K,)),
            pltpu.SemaphoreType.DMA((K,)),
            pltpu.SemaphoreType.DMA((K,)),
            pltpu.SemaphoreType.DMA((3, K)),
        ],
        compiler_params=pltpu.CompilerParams(collective_id=0),
    )(local, counts, mask, E)
